# Initial kernel scaffold; baseline (speedup 1.0000x reference)
import jax
import jax.numpy as jnp
from jax.experimental import pallas as pl


def kernel(atomic_numbers, W):
    raise NotImplementedError("write your pallas kernel here")


# speedup vs baseline: 5.7128x; 5.7128x over previous
import functools

import jax
import jax.numpy as jnp
from jax import lax
from jax.experimental import pallas as pl
from jax.experimental.pallas import tpu as pltpu
from jax.experimental.pallas import tpu_sc as plsc

B, SEQ = 16384, 200
N_IDX = B * SEQ
ROWS, COLS = 100, 8
NC, NS, L = 2, 16, 16
NW = NC * NS
PER_W = N_IDX // NW
CHUNK = 6400
N_CHUNKS = PER_W // CHUNK
VECS = CHUNK // L


def _lookup_body(idx_hbm, w_hbm, out_hbm, w_v, idx_v, out_v):
    wid = lax.axis_index("s") * NC + lax.axis_index("c")
    my_base = wid * PER_W
    pltpu.sync_copy(w_hbm, w_v)
    iota = lax.iota(jnp.int32, L)
    iota8 = iota * COLS

    def chunk_body(k, carry):
        cbase = my_base + k * CHUNK

        pltpu.sync_copy(idx_hbm.at[pl.ds(cbase, CHUNK)], idx_v)

        def vec_body(j, c2):
            idxv = idx_v[pl.ds(j * L, L)]
            g = idxv * COLS
            s0 = j * (L * COLS)
            for c in range(COLS):
                vals = plsc.load_gather(w_v, [g + c])
                plsc.store_scatter(out_v, [iota8 + (s0 + c)], vals)
            return c2

        lax.fori_loop(0, VECS, vec_body, 0)
        pltpu.sync_copy(out_v, out_hbm.at[pl.ds(cbase * COLS, CHUNK * COLS)])
        return carry

    lax.fori_loop(0, N_CHUNKS, chunk_body, 0)


_lookup = functools.partial(
    pl.kernel,
    out_type=jax.ShapeDtypeStruct((N_IDX * COLS,), jnp.float32),
    mesh=plsc.VectorSubcoreMesh(core_axis_name="c", subcore_axis_name="s"),
    compiler_params=pltpu.CompilerParams(needs_layout_passes=False),
    scratch_types=[
        pltpu.VMEM((ROWS * COLS,), jnp.float32),
        pltpu.VMEM((CHUNK,), jnp.int32),
        pltpu.VMEM((CHUNK * COLS,), jnp.float32),
    ],
)(_lookup_body)


def kernel(atomic_numbers, W):
    idx = atomic_numbers.reshape(-1)
    out = _lookup(idx, W.reshape(-1))
    return out.reshape(B, SEQ, COLS)


# speedup vs baseline: 6.5372x; 1.1443x over previous
import functools

import jax
import jax.numpy as jnp
from jax import lax
from jax.experimental import pallas as pl
from jax.experimental.pallas import tpu as pltpu
from jax.experimental.pallas import tpu_sc as plsc

B, SEQ = 16384, 200
N_IDX = B * SEQ
ROWS, COLS = 100, 8
NC, NS, L = 2, 16, 16
NW = NC * NS
PER_W = N_IDX // NW
CHUNK = 6400
N_CHUNKS = PER_W // CHUNK
VECS = CHUNK // L


def _lookup_body(idx_hbm, w_hbm, out_hbm, w_v, idx_v, out_v):
    wid = lax.axis_index("s") * NC + lax.axis_index("c")
    my_base = wid * PER_W
    pltpu.sync_copy(w_hbm, w_v)
    iota = lax.iota(jnp.int32, L)
    iota8 = iota * COLS

    def chunk_body(k, carry):
        cbase = my_base + k * CHUNK

        pltpu.sync_copy(idx_hbm.at[pl.ds(cbase, CHUNK)], idx_v)

        @plsc.parallel_loop(0, VECS, unroll=4)
        def vec_body(j):
            idxv = idx_v[pl.ds(j * L, L)]
            g = idxv * COLS
            s0 = j * (L * COLS)
            for c in range(COLS):
                vals = plsc.load_gather(w_v, [g + c])
                plsc.store_scatter(out_v, [iota8 + (s0 + c)], vals)
        pltpu.sync_copy(out_v, out_hbm.at[pl.ds(cbase * COLS, CHUNK * COLS)])
        return carry

    lax.fori_loop(0, N_CHUNKS, chunk_body, 0)


_lookup = functools.partial(
    pl.kernel,
    out_type=jax.ShapeDtypeStruct((N_IDX * COLS,), jnp.float32),
    mesh=plsc.VectorSubcoreMesh(core_axis_name="c", subcore_axis_name="s"),
    compiler_params=pltpu.CompilerParams(needs_layout_passes=False),
    scratch_types=[
        pltpu.VMEM((ROWS * COLS,), jnp.float32),
        pltpu.VMEM((CHUNK,), jnp.int32),
        pltpu.VMEM((CHUNK * COLS,), jnp.float32),
    ],
)(_lookup_body)


def kernel(atomic_numbers, W):
    idx = atomic_numbers.reshape(-1)
    out = _lookup(idx, W.reshape(-1))
    return out.reshape(B, SEQ, COLS)


# speedup vs baseline: 6.6709x; 1.0205x over previous
import functools

import jax
import jax.numpy as jnp
from jax import lax
from jax.experimental import pallas as pl
from jax.experimental.pallas import tpu as pltpu
from jax.experimental.pallas import tpu_sc as plsc

B, SEQ = 16384, 200
N_IDX = B * SEQ
ROWS, COLS = 100, 8
NC, NS, L = 2, 16, 16
NW = NC * NS
PER_W = N_IDX // NW
CHUNK = 6400
N_CHUNKS = PER_W // CHUNK
VECS = CHUNK // L


def _lookup_body(idx_hbm, w_hbm, out_hbm, w_v, idx_v, out_v):
    wid = lax.axis_index("s") * NC + lax.axis_index("c")
    my_base = wid * PER_W
    pltpu.sync_copy(w_hbm, w_v)
    iota = lax.iota(jnp.int32, L)
    iota8 = iota * COLS
    half = iota >> 1
    qs = [(c + half) & (COLS - 1) for c in range(COLS)]
    s_base = [iota8 + qs[c] for c in range(COLS)]

    def chunk_body(k, carry):
        cbase = my_base + k * CHUNK

        pltpu.sync_copy(idx_hbm.at[pl.ds(cbase, CHUNK)], idx_v)

        @plsc.parallel_loop(0, VECS, unroll=4)
        def vec_body(j):
            idxv = idx_v[pl.ds(j * L, L)]
            g = idxv * COLS
            s0 = j * (L * COLS)
            for c in range(COLS):
                vals = plsc.load_gather(w_v, [g + qs[c]])
                plsc.store_scatter(out_v, [s_base[c] + s0], vals)
        pltpu.sync_copy(out_v, out_hbm.at[pl.ds(cbase * COLS, CHUNK * COLS)])
        return carry

    lax.fori_loop(0, N_CHUNKS, chunk_body, 0)


_lookup = functools.partial(
    pl.kernel,
    out_type=jax.ShapeDtypeStruct((N_IDX * COLS,), jnp.float32),
    mesh=plsc.VectorSubcoreMesh(core_axis_name="c", subcore_axis_name="s"),
    compiler_params=pltpu.CompilerParams(needs_layout_passes=False),
    scratch_types=[
        pltpu.VMEM((ROWS * COLS,), jnp.float32),
        pltpu.VMEM((CHUNK,), jnp.int32),
        pltpu.VMEM((CHUNK * COLS,), jnp.float32),
    ],
)(_lookup_body)


def kernel(atomic_numbers, W):
    idx = atomic_numbers.reshape(-1)
    out = _lookup(idx, W.reshape(-1))
    return out.reshape(B, SEQ, COLS)


# speedup vs baseline: 97.8501x; 14.6681x over previous
import functools

import jax
import jax.numpy as jnp
from jax import lax
from jax.experimental import pallas as pl
from jax.experimental.pallas import tpu as pltpu
from jax.experimental.pallas import tpu_sc as plsc

B, SEQ = 16384, 200
N_IDX = B * SEQ
ROWS, COLS = 100, 8
NC, NS, L = 2, 16, 16
NW = NC * NS
QUARTER = B // 4
N_UNITS = SEQ * 4
UNITS_PER_W = N_UNITS // NW
VECS = QUARTER // L
OUT_UNIT = QUARTER * COLS


def _lookup_body(idx_hbm, wt_hbm, out_hbm, wt_v, idx_v, out_v):
    wid = lax.axis_index("s") * NC + lax.axis_index("c")
    pltpu.sync_copy(wt_hbm, wt_v)

    def unit_body(u, carry):
        gid = u * NW + wid
        in_base = gid * QUARTER
        out_base = gid * OUT_UNIT

        pltpu.sync_copy(idx_hbm.at[pl.ds(in_base, QUARTER)], idx_v)

        @plsc.parallel_loop(0, VECS, unroll=8)
        def vec_body(v):
            idxv = idx_v[pl.ds(v * L, L)]
            bt = v >> 3
            g = v & 7
            o0 = bt * (COLS * 128) + g * L
            for c in range(COLS):
                vals = plsc.load_gather(wt_v, [idxv + (c * ROWS)])
                out_v[pl.ds(o0 + c * 128, L)] = vals

        pltpu.sync_copy(out_v, out_hbm.at[pl.ds(out_base, OUT_UNIT)])
        return carry

    lax.fori_loop(0, UNITS_PER_W, unit_body, 0)


_lookup = functools.partial(
    pl.kernel,
    out_type=jax.ShapeDtypeStruct((N_IDX * COLS,), jnp.float32),
    mesh=plsc.VectorSubcoreMesh(core_axis_name="c", subcore_axis_name="s"),
    compiler_params=pltpu.CompilerParams(needs_layout_passes=False),
    scratch_types=[
        pltpu.VMEM((ROWS * COLS,), jnp.float32),
        pltpu.VMEM((QUARTER,), jnp.int32),
        pltpu.VMEM((OUT_UNIT,), jnp.float32),
    ],
)(_lookup_body)


def kernel(atomic_numbers, W):
    idx_t = atomic_numbers.T.reshape(-1)
    wt = W.T.reshape(-1)
    flat = _lookup(idx_t, wt)
    out4 = flat.reshape(SEQ, B // 128, COLS, 128)
    return out4.transpose(1, 3, 0, 2).reshape(B, SEQ, COLS)


# speedup vs baseline: 158.3202x; 1.6180x over previous
import functools

import jax
import jax.numpy as jnp
from jax import lax
from jax.experimental import pallas as pl
from jax.experimental.pallas import tpu as pltpu
from jax.experimental.pallas import tpu_sc as plsc

B, SEQ = 16384, 200
N_IDX = B * SEQ
ROWS, COLS = 100, 8
NC, NS, L = 2, 16, 16
NW = NC * NS
QUARTER = B // 4
N_UNITS = SEQ * 4
UNITS_PER_W = N_UNITS // NW
VECS = QUARTER // L
OUT_UNIT = QUARTER * COLS


def _lookup_body(idx_hbm, wt_hbm, out_hbm, wt_v,
                 idx_v0, idx_v1, out_v0, out_v1,
                 sin0, sin1, sout0, sout1):
    wid = lax.axis_index("s") * NC + lax.axis_index("c")
    pltpu.sync_copy(wt_hbm, wt_v)

    ibufs, obufs = (idx_v0, idx_v1), (out_v0, out_v1)
    sins, souts = (sin0, sin1), (sout0, sout1)

    def in_slice(u):
        return idx_hbm.at[pl.ds((u * NW + wid) * QUARTER, QUARTER)]

    def out_slice(u):
        return out_hbm.at[pl.ds((u * NW + wid) * OUT_UNIT, OUT_UNIT)]

    def compute(idx_v, out_v):
        @plsc.parallel_loop(0, VECS, unroll=8)
        def vec_body(v):
            idxv = idx_v[pl.ds(v * L, L)]
            bt = v >> 3
            g = v & 7
            o0 = bt * (COLS * 128) + g * L
            for c in range(COLS):
                vals = plsc.load_gather(wt_v, [idxv + (c * ROWS)])
                out_v[pl.ds(o0 + c * 128, L)] = vals

    pltpu.async_copy(in_slice(0), ibufs[0], sins[0])
    for u in (0, 1):
        pltpu.async_copy(in_slice(u + 1), ibufs[1 - u], sins[1 - u])
        pltpu.make_async_copy(in_slice(u), ibufs[u], sins[u]).wait()
        compute(ibufs[u], obufs[u])
        pltpu.async_copy(obufs[u], out_slice(u), souts[u])

    def pair_body(uu, carry):
        for b in range(2):
            u = uu * 2 + b
            pltpu.async_copy(in_slice(u + 1), ibufs[1 - b], sins[1 - b])
            pltpu.make_async_copy(in_slice(u), ibufs[b], sins[b]).wait()
            pltpu.make_async_copy(obufs[b], out_slice(u - 2), souts[b]).wait()
            compute(ibufs[b], obufs[b])
            pltpu.async_copy(obufs[b], out_slice(u), souts[b])
        return carry

    lax.fori_loop(1, 12, pair_body, 0)

    u_last = UNITS_PER_W - 1
    pltpu.make_async_copy(in_slice(u_last), ibufs[0], sins[0]).wait()
    pltpu.make_async_copy(obufs[0], out_slice(u_last - 2), souts[0]).wait()
    compute(ibufs[0], obufs[0])
    pltpu.async_copy(obufs[0], out_slice(u_last), souts[0])

    pltpu.make_async_copy(obufs[1], out_slice(u_last - 1), souts[1]).wait()
    pltpu.make_async_copy(obufs[0], out_slice(u_last), souts[0]).wait()


_lookup = functools.partial(
    pl.kernel,
    out_type=jax.ShapeDtypeStruct((N_IDX * COLS,), jnp.float32),
    mesh=plsc.VectorSubcoreMesh(core_axis_name="c", subcore_axis_name="s"),
    compiler_params=pltpu.CompilerParams(needs_layout_passes=False),
    scratch_types=[
        pltpu.VMEM((ROWS * COLS,), jnp.float32),
        pltpu.VMEM((QUARTER,), jnp.int32),
        pltpu.VMEM((QUARTER,), jnp.int32),
        pltpu.VMEM((OUT_UNIT,), jnp.float32),
        pltpu.VMEM((OUT_UNIT,), jnp.float32),
        pltpu.SemaphoreType.DMA,
        pltpu.SemaphoreType.DMA,
        pltpu.SemaphoreType.DMA,
        pltpu.SemaphoreType.DMA,
    ],
)(_lookup_body)


def kernel(atomic_numbers, W):
    idx_t = atomic_numbers.T.reshape(-1)
    wt = W.T.reshape(-1)
    flat = _lookup(idx_t, wt)
    out4 = flat.reshape(SEQ, B // 128, COLS, 128)
    return out4.transpose(1, 3, 0, 2).reshape(B, SEQ, COLS)
